# baseline (speedup 1.0000x reference)
import functools

import jax
import jax.numpy as jnp
from jax import lax
from jax.experimental import pallas as pl
from jax.experimental.pallas import tpu as pltpu
from jax.experimental.pallas import tpu_sc as plsc

NUM_CORES = 2
NUM_SUBCORES = 16
NUM_WORKERS = NUM_CORES * NUM_SUBCORES
BATCH = 16384
EMBED_DIM = 64
NUM_ENT = 1000000
BPW = BATCH // NUM_WORKERS
CHUNK = 128
NCHUNK = BPW // CHUNK
IDX_ROWS_PER_W = BPW // CHUNK


def _sc_body(head_h, rel_h, tail_h, entf_h, relemb_h, out_h,
             hidx, ridx, tidx, gidx, hrows, rrows, trows, outv, sem):
    wid = lax.axis_index("s") * NUM_CORES + lax.axis_index("c")
    rbase = wid * IDX_ROWS_PER_W

    pltpu.sync_copy(head_h.at[pl.ds(rbase, IDX_ROWS_PER_W)], hidx)
    pltpu.sync_copy(rel_h.at[pl.ds(rbase, IDX_ROWS_PER_W)], ridx)
    pltpu.sync_copy(tail_h.at[pl.ds(rbase, IDX_ROWS_PER_W)], tidx)

    lanes = lax.iota(jnp.int32, 16)
    dnums = lax.GatherDimensionNumbers(
        offset_dims=(), collapsed_slice_dims=(0,), start_index_map=(0,))

    def lane_sum(v):
        for s in (8, 4, 2, 1):
            perm = lax.gather(
                v, (lanes ^ s)[:, None], dimension_numbers=dnums,
                slice_sizes=(1,),
                mode=lax.GatherScatterMode.PROMISE_IN_BOUNDS)
            v = v + perm
        return v

    def fill_indices(idx_ref, plane, j, n_rows):
        dconst = [(c * 16 + lanes) * n_rows for c in range(EMBED_DIM // 16)]

        def grp(g, carry):
            ev = idx_ref[j, pl.ds(g * 16, 16)]
            for k in range(16):
                e = ev[k]
                p = g * 8 + k // 2
                off = (k % 2) * EMBED_DIM
                for c in range(EMBED_DIM // 16):
                    gidx[plane, p, pl.ds(off + c * 16, 16)] = dconst[c] + e
            return carry

        lax.fori_loop(0, CHUNK // 16, grp, 0)

    def per_chunk(j, carry):
        fill_indices(hidx, 0, j, NUM_ENT)
        fill_indices(ridx, 1, j, 1000)
        fill_indices(tidx, 2, j, NUM_ENT)

        def issue(r, c2):
            pltpu.async_copy(entf_h.at[gidx.at[0, r]],
                             hrows.at[pl.ds(r * CHUNK, CHUNK)], sem)
            pltpu.async_copy(relemb_h.at[gidx.at[1, r]],
                             rrows.at[pl.ds(r * CHUNK, CHUNK)], sem)
            pltpu.async_copy(entf_h.at[gidx.at[2, r]],
                             trows.at[pl.ds(r * CHUNK, CHUNK)], sem)
            return c2

        lax.fori_loop(0, CHUNK // 2, issue, 0)

        def drain(r, c2):
            for _ in range(3):
                pltpu.make_async_copy(
                    entf_h.at[pl.ds(0, CHUNK)],
                    hrows.at[pl.ds(0, CHUNK)], sem).wait()
            return c2

        lax.fori_loop(0, CHUNK // 2, drain, 0)

        def group(g, carry2):
            base = g * 16
            scores = jnp.zeros((16,), jnp.float32)
            for k in range(16):
                b = (base + k) * EMBED_DIM
                acc = (hrows[pl.ds(b, 16)] * rrows[pl.ds(b, 16)]
                       * trows[pl.ds(b, 16)])
                for c in range(1, EMBED_DIM // 16):
                    acc = acc + (hrows[pl.ds(b + c * 16, 16)]
                                 * rrows[pl.ds(b + c * 16, 16)]
                                 * trows[pl.ds(b + c * 16, 16)])
                scores = jnp.where(lanes == k, lane_sum(acc), scores)
            outv[pl.ds(j * CHUNK + base, 16)] = scores
            return carry2

        lax.fori_loop(0, CHUNK // 16, group, 0)
        return carry

    lax.fori_loop(0, NCHUNK, per_chunk, 0)

    pltpu.sync_copy(outv, out_h.at[pl.ds(wid * BPW, BPW)])


@jax.jit
def kernel(head, relation, tail, entity_embeddings, relation_embeddings):
    h = head.astype(jnp.int32).reshape(BATCH // CHUNK, CHUNK)
    r = relation.astype(jnp.int32).reshape(BATCH // CHUNK, CHUNK)
    t = tail.astype(jnp.int32).reshape(BATCH // CHUNK, CHUNK)

    entf = entity_embeddings.T.reshape(-1)
    relf = relation_embeddings.T.reshape(-1)

    mesh = plsc.VectorSubcoreMesh(core_axis_name="c", subcore_axis_name="s")
    run = functools.partial(
        pl.kernel,
        mesh=mesh,
        compiler_params=pltpu.CompilerParams(use_tc_tiling_on_sc=False),
        out_type=jax.ShapeDtypeStruct((BATCH,), jnp.float32),
        scratch_types=[
            pltpu.VMEM((IDX_ROWS_PER_W, CHUNK), jnp.int32),
            pltpu.VMEM((IDX_ROWS_PER_W, CHUNK), jnp.int32),
            pltpu.VMEM((IDX_ROWS_PER_W, CHUNK), jnp.int32),
            pltpu.VMEM((3, EMBED_DIM, CHUNK), jnp.int32),
            pltpu.VMEM((CHUNK * EMBED_DIM,), jnp.float32),
            pltpu.VMEM((CHUNK * EMBED_DIM,), jnp.float32),
            pltpu.VMEM((CHUNK * EMBED_DIM,), jnp.float32),
            pltpu.VMEM((BPW,), jnp.float32),
            pltpu.SemaphoreType.DMA,
        ],
    )(_sc_body)
    return run(h, r, t, entf, relf)


# speedup vs baseline: 8.2001x; 8.2001x over previous
import functools

import jax
import jax.numpy as jnp
from jax import lax
from jax.experimental import pallas as pl
from jax.experimental.pallas import tpu as pltpu
from jax.experimental.pallas import tpu_sc as plsc

NUM_CORES = 2
NUM_SUBCORES = 16
NUM_WORKERS = NUM_CORES * NUM_SUBCORES
BATCH = 16384
EMBED_DIM = 64
BPW = BATCH // NUM_WORKERS
CHUNK = 128
NCHUNK = BPW // CHUNK
IDX_ROWS_PER_W = BPW // CHUNK


def _sc_body(head_h, rel_h, tail_h, ent_h, relemb_h, out_h,
             hidx, ridx, tidx, hrows, rrows, trows, outv, sem):
    wid = lax.axis_index("s") * NUM_CORES + lax.axis_index("c")
    rbase = wid * IDX_ROWS_PER_W

    pltpu.sync_copy(head_h.at[pl.ds(rbase, IDX_ROWS_PER_W)], hidx)
    pltpu.sync_copy(rel_h.at[pl.ds(rbase, IDX_ROWS_PER_W)], ridx)
    pltpu.sync_copy(tail_h.at[pl.ds(rbase, IDX_ROWS_PER_W)], tidx)

    cps = []
    for j in range(NCHUNK):
        cps.append(pltpu.async_copy(
            ent_h.at[hidx.at[j]], hrows.at[pl.ds(j * CHUNK, CHUNK)], sem))
        cps.append(pltpu.async_copy(
            relemb_h.at[ridx.at[j]], rrows.at[pl.ds(j * CHUNK, CHUNK)], sem))
        cps.append(pltpu.async_copy(
            ent_h.at[tidx.at[j]], trows.at[pl.ds(j * CHUNK, CHUNK)], sem))
    for cp in cps:
        cp.wait()

    lanes = lax.iota(jnp.int32, 16)
    dnums = lax.GatherDimensionNumbers(
        offset_dims=(), collapsed_slice_dims=(0,), start_index_map=(0,))

    def lane_sum(v):
        for s in (8, 4, 2, 1):
            perm = lax.gather(
                v, (lanes ^ s)[:, None], dimension_numbers=dnums,
                slice_sizes=(1,),
                mode=lax.GatherScatterMode.PROMISE_IN_BOUNDS)
            v = v + perm
        return v

    def group(g, carry):
        base = g * 16
        scores = jnp.zeros((16,), jnp.float32)
        for j in range(16):
            b = base + j
            acc = (hrows[b, pl.ds(0, 16)] * rrows[b, pl.ds(0, 16)]
                   * trows[b, pl.ds(0, 16)])
            for c in range(1, EMBED_DIM // 16):
                acc = acc + (hrows[b, pl.ds(c * 16, 16)]
                             * rrows[b, pl.ds(c * 16, 16)]
                             * trows[b, pl.ds(c * 16, 16)])
            scores = jnp.where(lanes == j, lane_sum(acc), scores)
        outv[pl.ds(base, 16)] = scores
        return carry

    lax.fori_loop(0, BPW // 16, group, 0)

    pltpu.sync_copy(outv, out_h.at[pl.ds(wid * BPW, BPW)])


@jax.jit
def kernel(head, relation, tail, entity_embeddings, relation_embeddings):
    h = head.astype(jnp.int32).reshape(BATCH // CHUNK, CHUNK)
    r = relation.astype(jnp.int32).reshape(BATCH // CHUNK, CHUNK)
    t = tail.astype(jnp.int32).reshape(BATCH // CHUNK, CHUNK)

    mesh = plsc.VectorSubcoreMesh(core_axis_name="c", subcore_axis_name="s")
    run = functools.partial(
        pl.kernel,
        mesh=mesh,
        compiler_params=pltpu.CompilerParams(use_tc_tiling_on_sc=False),
        out_type=jax.ShapeDtypeStruct((BATCH,), jnp.float32),
        scratch_types=[
            pltpu.VMEM((IDX_ROWS_PER_W, CHUNK), jnp.int32),
            pltpu.VMEM((IDX_ROWS_PER_W, CHUNK), jnp.int32),
            pltpu.VMEM((IDX_ROWS_PER_W, CHUNK), jnp.int32),
            pltpu.VMEM((BPW, EMBED_DIM), jnp.float32),
            pltpu.VMEM((BPW, EMBED_DIM), jnp.float32),
            pltpu.VMEM((BPW, EMBED_DIM), jnp.float32),
            pltpu.VMEM((BPW,), jnp.float32),
            pltpu.SemaphoreType.DMA,
        ],
    )(_sc_body)
    return run(h, r, t, entity_embeddings, relation_embeddings)
